# baseline (speedup 1.0000x reference)
import functools

import jax
import jax.numpy as jnp
from jax import lax
from jax.experimental import pallas as pl
from jax.experimental.pallas import tpu as pltpu
from jax.experimental.pallas import tpu_sc as plsc

NUM_CORES = 2
NUM_SUBCORES = 16
NUM_WORKERS = NUM_CORES * NUM_SUBCORES
LANES = 16
CHUNK = 640


def _sc_lookup_concat(flat_tables, i2, i3, i4, tokens, num_tables, vocab, dim):
    heads = i2.shape[1]
    per_w = tokens // NUM_WORKERS
    n_chunks = per_w // CHUNK
    mesh = plsc.VectorSubcoreMesh(core_axis_name="c", subcore_axis_name="s")

    @functools.partial(
        pl.kernel,
        mesh=mesh,
        out_type=jax.ShapeDtypeStruct((tokens, num_tables, dim), jnp.float32),
        compiler_params=pltpu.CompilerParams(
            use_tc_tiling_on_sc=False, needs_layout_passes=False),
        scratch_types=[
            pltpu.VMEM((3, CHUNK, heads), jnp.int32),
            pltpu.VMEM((2, num_tables, CHUNK), jnp.int32),
            pltpu.VMEM((2, CHUNK, dim), jnp.float32),
            pltpu.SemaphoreType.DMA,
            pltpu.SemaphoreType.DMA,
            pltpu.SemaphoreType.DMA,
            pltpu.SemaphoreType.DMA,
        ],
    )
    def k(tab_hbm, i2_hbm, i3_hbm, i4_hbm, out_hbm, blk_v, idx_v, rows_v,
          g0, g1, w0, w1):
        gsem = (g0, g1)
        wsem = (w0, w1)
        idx_srcs = (i2_hbm, i3_hbm, i4_hbm)
        wid = lax.axis_index("s") * NUM_CORES + lax.axis_index("c")
        base0 = wid * per_w

        def phase_a(c, cb):
            base = base0 + c * CHUNK
            for n in range(3):
                pltpu.sync_copy(idx_srcs[n].at[pl.ds(base, CHUNK)], blk_v.at[n])
            for n in range(3):
                for h in range(heads):
                    t = n * heads + h

                    @pl.loop(0, CHUNK // LANES)
                    def _(j, n=n, h=h, t=t):
                        rows16 = plsc.load_gather(
                            blk_v.at[n],
                            [j * LANES + lax.iota(jnp.int32, 16),
                             jnp.full((16,), h, jnp.int32)],
                        )
                        idx_v[cb, t, pl.ds(j * LANES, LANES)] = rows16 + t * vocab

        def start_gather(c, t, b, cb):
            pltpu.async_copy(tab_hbm.at[idx_v.at[cb, t]], rows_v.at[b], gsem[b])

        def wait_gather(b):
            pltpu.make_async_copy(
                tab_hbm.at[idx_v.at[0, 0]], rows_v.at[b], gsem[b]).wait()

        def start_write(c, t, b):
            base = base0 + c * CHUNK
            pltpu.async_copy(rows_v.at[b], out_hbm.at[pl.ds(base, CHUNK), t], wsem[b])

        def wait_write(b):
            pltpu.make_async_copy(
                rows_v.at[b], out_hbm.at[pl.ds(base0, CHUNK), 0], wsem[b]).wait()

        phase_a(0, 0)
        start_gather(0, 0, 0, 0)

        @pl.loop(0, n_chunks // 2)
        def _(i):
            for cslot in range(2):
                cb = cslot
                c = 2 * i + cslot
                for t in range(num_tables):
                    b = t % 2

                    if cslot == 0 and t == 0:
                        @pl.when(i > 0)
                        def _():
                            wait_write(b ^ 1)
                    else:
                        wait_write(b ^ 1)

                    if t < num_tables - 1:
                        start_gather(c, t + 1, b ^ 1, cb)
                    elif cslot == 0:
                        phase_a(c + 1, cb ^ 1)
                        start_gather(c + 1, 0, b ^ 1, cb ^ 1)
                    else:
                        @pl.when(i < n_chunks // 2 - 1)
                        def _():
                            phase_a(c + 1, cb ^ 1)
                            start_gather(c + 1, 0, b ^ 1, cb ^ 1)

                    wait_gather(b)
                    start_write(c, t, b)

        wait_write(1)

    return k(flat_tables, i2, i3, i4)


def kernel(indices_2, indices_3, indices_4, tables):
    batch, seq, heads = indices_2.shape
    num_tables, vocab, dim = tables.shape
    tokens = batch * seq
    out = _sc_lookup_concat(
        tables.reshape(num_tables * vocab, dim),
        indices_2.reshape(tokens, heads),
        indices_3.reshape(tokens, heads),
        indices_4.reshape(tokens, heads),
        tokens, num_tables, vocab, dim,
    )
    return out.reshape(batch, seq, num_tables * dim)


# speedup vs baseline: 1.0034x; 1.0034x over previous
import functools

import jax
import jax.numpy as jnp
from jax import lax
from jax.experimental import pallas as pl
from jax.experimental.pallas import tpu as pltpu
from jax.experimental.pallas import tpu_sc as plsc

NUM_CORES = 2
NUM_SUBCORES = 16
NUM_WORKERS = NUM_CORES * NUM_SUBCORES
LANES = 16
CROWS = 8


def _sc_lookup_concat(tables, i2_raw, i3_raw, i4_raw):
    num_tables, vocab, dim = tables.shape
    batch, seq, heads = i2_raw.shape
    chunk = CROWS * seq
    rows_per_w = batch // NUM_WORKERS
    n_chunks = rows_per_w // CROWS
    mesh = plsc.VectorSubcoreMesh(core_axis_name="c", subcore_axis_name="s")

    @functools.partial(
        pl.kernel,
        mesh=mesh,
        out_type=jax.ShapeDtypeStruct((batch, seq, num_tables * dim), jnp.float32),
        compiler_params=pltpu.CompilerParams(
            use_tc_tiling_on_sc=False, needs_layout_passes=False),
        scratch_types=[
            pltpu.VMEM((3, CROWS, seq, heads), jnp.int32),
            pltpu.VMEM((2, num_tables, chunk), jnp.int32),
            pltpu.VMEM((2, chunk, dim), jnp.float32),
            pltpu.SemaphoreType.DMA,
            pltpu.SemaphoreType.DMA,
            pltpu.SemaphoreType.DMA,
            pltpu.SemaphoreType.DMA,
        ],
    )
    def k(tab_hbm, i2_hbm, i3_hbm, i4_hbm, out_hbm, blk_v, idx_v, rows_v,
          g0, g1, w0, w1):
        gsem = (g0, g1)
        wsem = (w0, w1)
        idx_srcs = (i2_hbm, i3_hbm, i4_hbm)
        wid = lax.axis_index("s") * NUM_CORES + lax.axis_index("c")
        brow0 = wid * rows_per_w

        def phase_a(c, cb):
            brow = brow0 + c * CROWS
            for n in range(3):
                pltpu.sync_copy(idx_srcs[n].at[pl.ds(brow, CROWS)], blk_v.at[n])

            @pl.loop(0, chunk // LANES)
            def _(j):
                tok = j * LANES + lax.iota(jnp.int32, 16)
                d0 = tok // seq
                d1 = tok - d0 * seq
                for n in range(3):
                    for h in range(heads):
                        t = n * heads + h
                        rows16 = plsc.load_gather(
                            blk_v.at[n], [d0, d1, jnp.full((16,), h, jnp.int32)])
                        idx_v[cb, t, pl.ds(j * LANES, LANES)] = rows16

        def start_gather(t, b, cb):
            pltpu.async_copy(tab_hbm.at[t].at[idx_v.at[cb, t]], rows_v.at[b],
                             gsem[b])

        def wait_gather(b):
            pltpu.make_async_copy(
                tab_hbm.at[0].at[idx_v.at[0, 0]], rows_v.at[b], gsem[b]).wait()

        def start_write(c, t, b):
            brow = brow0 + c * CROWS
            for r in range(CROWS):
                pltpu.async_copy(
                    rows_v.at[b, pl.ds(r * seq, seq)],
                    out_hbm.at[brow + r, :, pl.ds(t * dim, dim)], wsem[b])

        def wait_write(b):
            for _r in range(CROWS):
                pltpu.make_async_copy(
                    rows_v.at[b, pl.ds(0, seq)],
                    out_hbm.at[brow0, :, pl.ds(0, dim)], wsem[b]).wait()

        phase_a(0, 0)
        start_gather(0, 0, 0)

        @pl.loop(0, n_chunks // 2)
        def _(i):
            for cslot in range(2):
                cb = cslot
                c = 2 * i + cslot
                for t in range(num_tables):
                    b = t % 2

                    if cslot == 0 and t == 0:
                        @pl.when(i > 0)
                        def _():
                            wait_write(b ^ 1)
                    else:
                        wait_write(b ^ 1)

                    if t < num_tables - 1:
                        start_gather(t + 1, b ^ 1, cb)
                    elif cslot == 0:
                        phase_a(c + 1, cb ^ 1)
                        start_gather(0, b ^ 1, cb ^ 1)
                    else:
                        @pl.when(i < n_chunks // 2 - 1)
                        def _():
                            phase_a(c + 1, cb ^ 1)
                            start_gather(0, b ^ 1, cb ^ 1)

                    wait_gather(b)
                    start_write(c, t, b)

        wait_write(1)

    return k(tables, i2_raw, i3_raw, i4_raw)


def kernel(indices_2, indices_3, indices_4, tables):
    return _sc_lookup_concat(tables, indices_2, indices_3, indices_4)


# speedup vs baseline: 1.4830x; 1.4780x over previous
import functools

import jax
import jax.numpy as jnp
from jax import lax
from jax.experimental import pallas as pl
from jax.experimental.pallas import tpu as pltpu
from jax.experimental.pallas import tpu_sc as plsc

NUM_CORES = 2
NUM_SUBCORES = 16
NUM_WORKERS = NUM_CORES * NUM_SUBCORES
CHUNK = 640


def _sc_gather(flat_tables, idx, width):
    num_tables, tokens = idx.shape
    _, dim = flat_tables.shape
    per_w = tokens // NUM_WORKERS
    n_chunks = per_w // CHUNK
    n_units = n_chunks * num_tables
    mesh = plsc.VectorSubcoreMesh(core_axis_name="c", subcore_axis_name="s")

    @functools.partial(
        pl.kernel,
        mesh=mesh,
        out_type=jax.ShapeDtypeStruct((tokens, width), jnp.float32),
        compiler_params=pltpu.CompilerParams(use_tc_tiling_on_sc=False),
        scratch_types=[
            pltpu.VMEM((2, CHUNK), jnp.int32),
            pltpu.VMEM((2, CHUNK, dim), jnp.float32),
            pltpu.SemaphoreType.DMA,
            pltpu.SemaphoreType.DMA,
            pltpu.SemaphoreType.DMA,
            pltpu.SemaphoreType.DMA,
        ],
    )
    def k(tab_hbm, idx_hbm, out_hbm, idx_v, rows_v, g0, g1, w0, w1):
        gsem = (g0, g1)
        wsem = (w0, w1)
        wid = lax.axis_index("s") * NUM_CORES + lax.axis_index("c")
        base0 = wid * per_w

        def unit(u):
            return base0 + (u // num_tables) * CHUNK, u % num_tables

        def start_gather(u, b):
            base, t = unit(u)
            pltpu.sync_copy(idx_hbm.at[t, pl.ds(base, CHUNK)], idx_v.at[b])
            pltpu.async_copy(tab_hbm.at[idx_v.at[b]], rows_v.at[b], gsem[b])

        def wait_gather(b):
            pltpu.make_async_copy(tab_hbm.at[idx_v.at[b]], rows_v.at[b], gsem[b]).wait()

        def start_write(u, b):
            base, t = unit(u)
            pltpu.async_copy(
                rows_v.at[b],
                out_hbm.at[pl.ds(base, CHUNK), pl.ds(t * dim, dim)], wsem[b])

        def wait_write(b):
            pltpu.make_async_copy(
                rows_v.at[b],
                out_hbm.at[pl.ds(base0, CHUNK), pl.ds(0, dim)], wsem[b]).wait()

        start_gather(0, 0)

        @pl.loop(0, n_units // 2)
        def _(i):
            u = 2 * i

            @pl.when(i >= 1)
            def _():
                wait_write(1)

            start_gather(u + 1, 1)
            wait_gather(0)
            start_write(u, 0)

            @pl.when(i < n_units // 2 - 1)
            def _():
                wait_write(0)
                start_gather(u + 2, 0)

            wait_gather(1)
            start_write(u + 1, 1)

        wait_write(0)
        wait_write(1)

    return k(flat_tables, idx)


def kernel(indices_2, indices_3, indices_4, tables):
    batch, seq, heads = indices_2.shape
    num_tables, vocab, dim = tables.shape
    tokens = batch * seq
    idx = jnp.stack([indices_2, indices_3, indices_4], axis=0)
    idx = idx.transpose(0, 3, 2, 1).reshape(num_tables, tokens)
    idx = idx.astype(jnp.int32) + (jnp.arange(num_tables, dtype=jnp.int32) * vocab)[:, None]
    out = _sc_gather(tables.reshape(num_tables * vocab, dim), idx,
                     num_tables * dim)
    return out.reshape(seq, batch, num_tables * dim).transpose(1, 0, 2)


# speedup vs baseline: 1.8300x; 1.2340x over previous
import functools

import jax
import jax.numpy as jnp
from jax import lax
from jax.experimental import pallas as pl
from jax.experimental.pallas import tpu as pltpu
from jax.experimental.pallas import tpu_sc as plsc

NUM_CORES = 2
NUM_SUBCORES = 16
NUM_WORKERS = NUM_CORES * NUM_SUBCORES
CHUNK = 640
SUB = 8


def _sc_gather(flat_tables, idx, width):
    num_tables, tokens = idx.shape
    _, dim = flat_tables.shape
    per_w = tokens // NUM_WORKERS
    n_chunks = per_w // CHUNK
    n_units = n_chunks * num_tables
    t8 = CHUNK // SUB
    mesh = plsc.VectorSubcoreMesh(core_axis_name="c", subcore_axis_name="s")

    @functools.partial(
        pl.kernel,
        mesh=mesh,
        out_type=jax.ShapeDtypeStruct(
            (tokens // SUB, width // 128, SUB, 128), jnp.float32),
        compiler_params=pltpu.CompilerParams(use_tc_tiling_on_sc=False),
        scratch_types=[
            pltpu.VMEM((2, CHUNK), jnp.int32),
            pltpu.VMEM((2, CHUNK, dim), jnp.float32),
            pltpu.SemaphoreType.DMA,
            pltpu.SemaphoreType.DMA,
            pltpu.SemaphoreType.DMA,
            pltpu.SemaphoreType.DMA,
        ],
    )
    def k(tab_hbm, idx_hbm, out_hbm, idx_v, rows_v, g0, g1, w0, w1):
        gsem = (g0, g1)
        wsem = (w0, w1)
        wid = lax.axis_index("s") * NUM_CORES + lax.axis_index("c")
        base0 = wid * per_w

        def unit(u):
            return base0 + (u // num_tables) * CHUNK, u % num_tables

        def start_gather(u, b):
            base, t = unit(u)
            pltpu.sync_copy(idx_hbm.at[t, pl.ds(base, CHUNK)], idx_v.at[b])
            pltpu.async_copy(tab_hbm.at[idx_v.at[b]], rows_v.at[b], gsem[b])

        def wait_gather(b):
            pltpu.make_async_copy(tab_hbm.at[idx_v.at[b]], rows_v.at[b], gsem[b]).wait()

        def start_write(u, b):
            base, t = unit(u)
            blk = base // SUB
            for b8 in range(SUB):
                pltpu.async_copy(
                    rows_v.at[b, pl.ds(b8 * t8, t8)],
                    out_hbm.at[pl.ds(blk, t8), t // 2, b8,
                               pl.ds((t % 2) * dim, dim)],
                    wsem[b])

        def wait_write(b):
            for _ in range(SUB):
                pltpu.make_async_copy(
                    rows_v.at[b, pl.ds(0, t8)],
                    out_hbm.at[pl.ds(0, t8), 0, 0, pl.ds(0, dim)],
                    wsem[b]).wait()

        start_gather(0, 0)

        @pl.loop(0, n_units // 2)
        def _(i):
            u = 2 * i

            @pl.when(i >= 1)
            def _():
                wait_write(1)

            start_gather(u + 1, 1)
            wait_gather(0)
            start_write(u, 0)

            @pl.when(i < n_units // 2 - 1)
            def _():
                wait_write(0)
                start_gather(u + 2, 0)

            wait_gather(1)
            start_write(u + 1, 1)

        wait_write(0)
        wait_write(1)

    return k(flat_tables, idx)


def kernel(indices_2, indices_3, indices_4, tables):
    batch, seq, heads = indices_2.shape
    num_tables, vocab, dim = tables.shape
    tokens = batch * seq
    width = num_tables * dim
    idx = jnp.stack([indices_2, indices_3, indices_4], axis=0)
    idx = idx.transpose(0, 3, 2, 1).reshape(num_tables, tokens)
    idx = idx.astype(jnp.int32) + (jnp.arange(num_tables, dtype=jnp.int32) * vocab)[:, None]
    idx = (idx.reshape(num_tables, tokens // CHUNK, CHUNK // SUB, SUB)
           .transpose(0, 1, 3, 2).reshape(num_tables, tokens))
    out4 = _sc_gather(tables.reshape(num_tables * vocab, dim), idx, width)
    out = (out4.reshape(seq, batch // SUB, width // 128, SUB, 128)
           .transpose(0, 1, 3, 2, 4).reshape(seq, batch, width)
           .transpose(1, 0, 2))
    return out


# speedup vs baseline: 2.0778x; 1.1354x over previous
import functools

import jax
import jax.numpy as jnp
from jax import lax
from jax.experimental import pallas as pl
from jax.experimental.pallas import tpu as pltpu
from jax.experimental.pallas import tpu_sc as plsc

NUM_CORES = 2
NUM_SUBCORES = 16
NUM_WORKERS = NUM_CORES * NUM_SUBCORES
LANES = 16
CHUNK = 128
SUB = 8


def _sc_lookup_concat(flat_tables, i2p, i3p, i4p, vocab):
    _, dim = flat_tables.shape
    seq, bblocks, heads, lane = i2p.shape
    batch = bblocks * lane
    tokens = batch * seq
    num_tables = 3 * heads
    width = num_tables * dim
    per_w = tokens // NUM_WORKERS
    n_chunks = per_w // CHUNK
    t8 = CHUNK // SUB
    mesh = plsc.VectorSubcoreMesh(core_axis_name="c", subcore_axis_name="s")

    @functools.partial(
        pl.kernel,
        mesh=mesh,
        out_type=jax.ShapeDtypeStruct(
            (tokens // SUB, width // 128, SUB, 128), jnp.float32),
        compiler_params=pltpu.CompilerParams(
            use_tc_tiling_on_sc=False, needs_layout_passes=False),
        scratch_types=[
            pltpu.VMEM((3, heads, lane), jnp.int32),
            pltpu.VMEM((2, num_tables, CHUNK), jnp.int32),
            pltpu.VMEM((2, CHUNK, dim), jnp.float32),
            pltpu.SemaphoreType.DMA,
            pltpu.SemaphoreType.DMA,
            pltpu.SemaphoreType.DMA,
            pltpu.SemaphoreType.DMA,
        ],
    )
    def k(tab_hbm, i2_hbm, i3_hbm, i4_hbm, out_hbm, blk_v, idx_v, rows_v,
          g0, g1, w0, w1):
        gsem = (g0, g1)
        wsem = (w0, w1)
        idx_srcs = (i2_hbm, i3_hbm, i4_hbm)
        wid = lax.axis_index("s") * NUM_CORES + lax.axis_index("c")
        tb0 = wid * (per_w // SUB)

        def phase_a(c, cb):
            t8a = tb0 + c * t8
            s = t8a // (batch // SUB)
            b0 = t8a % (batch // SUB) // (lane // SUB)
            for n in range(3):
                pltpu.sync_copy(idx_srcs[n].at[s, b0], blk_v.at[n])
            for g in range(SUB):
                bl = lax.iota(jnp.int32, LANES) * SUB + g
                for n in range(3):
                    for h in range(heads):
                        t = n * heads + h
                        v = plsc.load_gather(
                            blk_v.at[n], [jnp.full((LANES,), h, jnp.int32), bl])
                        idx_v[cb, t, pl.ds(g * LANES, LANES)] = v + t * vocab

        def start_gather(t, b, cb):
            pltpu.async_copy(tab_hbm.at[idx_v.at[cb, t]], rows_v.at[b], gsem[b])

        def wait_gather(b):
            pltpu.make_async_copy(
                tab_hbm.at[idx_v.at[0, 0]], rows_v.at[b], gsem[b]).wait()

        def start_write(c, t, b):
            blk = tb0 + c * t8
            for b8 in range(SUB):
                pltpu.async_copy(
                    rows_v.at[b, pl.ds(b8 * t8, t8)],
                    out_hbm.at[pl.ds(blk, t8), t // 2, b8,
                               pl.ds((t % 2) * dim, dim)],
                    wsem[b])

        def wait_write(b):
            for _ in range(SUB):
                pltpu.make_async_copy(
                    rows_v.at[b, pl.ds(0, t8)],
                    out_hbm.at[pl.ds(0, t8), 0, 0, pl.ds(0, dim)],
                    wsem[b]).wait()

        phase_a(0, 0)
        start_gather(0, 0, 0)

        @pl.loop(0, n_chunks // 2)
        def _(i):
            for cslot in range(2):
                cb = cslot
                c = 2 * i + cslot
                for t in range(num_tables):
                    b = t % 2

                    if cslot == 0 and t == 0:
                        @pl.when(i > 0)
                        def _():
                            wait_write(b ^ 1)
                    else:
                        wait_write(b ^ 1)

                    if t < num_tables - 1:
                        start_gather(t + 1, b ^ 1, cb)
                    elif cslot == 0:
                        phase_a(c + 1, cb ^ 1)
                        start_gather(0, b ^ 1, cb ^ 1)
                    else:
                        @pl.when(i < n_chunks // 2 - 1)
                        def _():
                            phase_a(c + 1, cb ^ 1)
                            start_gather(0, b ^ 1, cb ^ 1)

                    wait_gather(b)
                    start_write(c, t, b)

        wait_write(1)

    return k(flat_tables, i2p, i3p, i4p)


def kernel(indices_2, indices_3, indices_4, tables):
    batch, seq, heads = indices_2.shape
    num_tables, vocab, dim = tables.shape
    width = num_tables * dim

    def as_tiles(ix):
        return (ix.astype(jnp.int32).transpose(1, 2, 0)
                .reshape(seq, heads, batch // 128, 128).transpose(0, 2, 1, 3))

    out4 = _sc_lookup_concat(
        tables.reshape(num_tables * vocab, dim),
        as_tiles(indices_2), as_tiles(indices_3), as_tiles(indices_4), vocab)
    out = (out4.reshape(seq, batch // SUB, width // 128, SUB, 128)
           .transpose(0, 1, 3, 2, 4).reshape(seq, batch, width)
           .transpose(1, 0, 2))
    return out


# speedup vs baseline: 2.2899x; 1.1021x over previous
import functools

import jax
import jax.numpy as jnp
from jax import lax
from jax.experimental import pallas as pl
from jax.experimental.pallas import tpu as pltpu
from jax.experimental.pallas import tpu_sc as plsc

NUM_CORES = 2
NUM_SUBCORES = 16
NUM_WORKERS = NUM_CORES * NUM_SUBCORES
LANES = 16
CHUNK = 256
SUB = 8


def _sc_lookup_concat(flat_tables, i2p, i3p, i4p, vocab):
    _, dim = flat_tables.shape
    seq, bblocks, heads, lane = i2p.shape
    batch = bblocks * lane
    tokens = batch * seq
    num_tables = 3 * heads
    width = num_tables * dim
    per_w = tokens // NUM_WORKERS
    n_chunks = per_w // CHUNK
    t8 = CHUNK // SUB
    mesh = plsc.VectorSubcoreMesh(core_axis_name="c", subcore_axis_name="s")

    @functools.partial(
        pl.kernel,
        mesh=mesh,
        out_type=jax.ShapeDtypeStruct(
            (tokens // SUB, width // 128, SUB, 128), jnp.float32),
        compiler_params=pltpu.CompilerParams(
            use_tc_tiling_on_sc=False, needs_layout_passes=False),
        scratch_types=[
            pltpu.VMEM((3, CHUNK // 128, heads, lane), jnp.int32),
            pltpu.VMEM((2, num_tables, CHUNK), jnp.int32),
            pltpu.VMEM((2, CHUNK, dim), jnp.float32),
            pltpu.SemaphoreType.DMA,
            pltpu.SemaphoreType.DMA,
            pltpu.SemaphoreType.DMA,
            pltpu.SemaphoreType.DMA,
        ],
    )
    def k(tab_hbm, i2_hbm, i3_hbm, i4_hbm, out_hbm, blk_v, idx_v, rows_v,
          g0, g1, w0, w1):
        gsem = (g0, g1)
        wsem = (w0, w1)
        idx_srcs = (i2_hbm, i3_hbm, i4_hbm)
        wid = lax.axis_index("s") * NUM_CORES + lax.axis_index("c")
        tb0 = wid * (per_w // SUB)

        def phase_a(c, cb):
            t8a = tb0 + c * t8
            s = t8a // (batch // SUB)
            b0 = t8a % (batch // SUB) // (lane // SUB)
            for n in range(3):
                pltpu.sync_copy(idx_srcs[n].at[s, pl.ds(b0, CHUNK // 128)],
                                blk_v.at[n])
            for g in range(CHUNK // LANES):
                i0 = jnp.full((LANES,), g % 2, jnp.int32)
                il = lax.iota(jnp.int32, LANES) * SUB + g // 2
                for n in range(3):
                    for h in range(heads):
                        t = n * heads + h
                        v = plsc.load_gather(
                            blk_v.at[n],
                            [i0, jnp.full((LANES,), h, jnp.int32), il])
                        idx_v[cb, t, pl.ds(g * LANES, LANES)] = v + t * vocab

        def start_gather(t, b, cb):
            pltpu.async_copy(tab_hbm.at[idx_v.at[cb, t]], rows_v.at[b], gsem[b])

        def wait_gather(b):
            pltpu.make_async_copy(
                tab_hbm.at[idx_v.at[0, 0]], rows_v.at[b], gsem[b]).wait()

        def start_write(c, t, b):
            blk = tb0 + c * t8
            for b8 in range(SUB):
                pltpu.async_copy(
                    rows_v.at[b, pl.ds(b8 * t8, t8)],
                    out_hbm.at[pl.ds(blk, t8), t // 2, b8,
                               pl.ds((t % 2) * dim, dim)],
                    wsem[b])

        def wait_write(b):
            for _ in range(SUB):
                pltpu.make_async_copy(
                    rows_v.at[b, pl.ds(0, t8)],
                    out_hbm.at[pl.ds(0, t8), 0, 0, pl.ds(0, dim)],
                    wsem[b]).wait()

        phase_a(0, 0)
        start_gather(0, 0, 0)

        @pl.loop(0, (n_chunks - 1) // 2)
        def _(i):
            for cslot in range(2):
                cb = cslot
                c = 2 * i + cslot
                for t in range(num_tables):
                    b = t % 2

                    if cslot == 0 and t == 0:
                        @pl.when(i > 0)
                        def _():
                            wait_write(b ^ 1)
                    else:
                        wait_write(b ^ 1)

                    if t < num_tables - 1:
                        start_gather(t + 1, b ^ 1, cb)
                    else:
                        phase_a(c + 1, cb ^ 1)
                        start_gather(0, b ^ 1, cb ^ 1)

                    wait_gather(b)
                    start_write(c, t, b)

        c_last = n_chunks - 1
        for t in range(num_tables):
            b = t % 2
            wait_write(b ^ 1)
            if t < num_tables - 1:
                start_gather(t + 1, b ^ 1, 0)
            wait_gather(b)
            start_write(c_last, t, b)

        wait_write(1)

    return k(flat_tables, i2p, i3p, i4p)


def kernel(indices_2, indices_3, indices_4, tables):
    batch, seq, heads = indices_2.shape
    num_tables, vocab, dim = tables.shape
    width = num_tables * dim

    def as_tiles(ix):
        return (ix.astype(jnp.int32).transpose(1, 2, 0)
                .reshape(seq, heads, batch // 128, 128).transpose(0, 2, 1, 3))

    out4 = _sc_lookup_concat(
        tables.reshape(num_tables * vocab, dim),
        as_tiles(indices_2), as_tiles(indices_3), as_tiles(indices_4), vocab)
    out = (out4.reshape(seq, batch // SUB, width // 128, SUB, 128)
           .transpose(0, 1, 3, 2, 4).reshape(seq, batch, width)
           .transpose(1, 0, 2))
    return out
